# Initial kernel scaffold; baseline (speedup 1.0000x reference)
import jax
import jax.numpy as jnp
from jax.experimental import pallas as pl


def kernel(x, edge_index, W1, a_src1, a_dst1, b1, W2, a_src2, a_dst2, b2):
    raise NotImplementedError("write your pallas kernel here")


# speedup vs baseline: 68.7517x; 68.7517x over previous
import functools

import jax
import jax.numpy as jnp
from jax import lax
from jax.experimental import pallas as pl
from jax.experimental.pallas import tpu as pltpu
from jax.experimental.pallas import tpu_sc as plsc

N = 10000
E = 320000
F_IN = 128
H1, C1 = 8, 8
NUM_CLASSES = 40

NC, NS = 2, 16
NW = NC * NS
K = 80
EPW = E // NW
NCHUNK = EPW // K
NPAD = 10240
RPT = NPAD // NS


def _take16(vec, idx):
    return lax.gather(
        vec, idx[:, None],
        lax.GatherDimensionNumbers(offset_dims=(), collapsed_slice_dims=(0,),
                                   start_index_map=(0,)),
        slice_sizes=(1,),
        mode=lax.GatherScatterMode.PROMISE_IN_BOUNDS,
    )


def _dense1_body(x_ref, w_ref, as_ref, ad_ref, ht_ref, ast_ref, adt_ref):
    h = jnp.dot(x_ref[...], w_ref[...], preferred_element_type=jnp.float32)
    als = jnp.dot(h, as_ref[...], preferred_element_type=jnp.float32)
    ald = jnp.dot(h, ad_ref[...], preferred_element_type=jnp.float32)
    ht_ref[...] = h
    ast_ref[...] = jnp.concatenate([als, als], axis=1)
    adt_ref[...] = jnp.concatenate([ald, ald], axis=1)


def _dense1(x, W1, As1, Ad1):
    blk = 1000
    return pl.pallas_call(
        _dense1_body,
        grid=(N // blk,),
        in_specs=[
            pl.BlockSpec((blk, F_IN), lambda i: (i, 0)),
            pl.BlockSpec((F_IN, H1 * C1), lambda i: (0, 0)),
            pl.BlockSpec((H1 * C1, H1), lambda i: (0, 0)),
            pl.BlockSpec((H1 * C1, H1), lambda i: (0, 0)),
        ],
        out_specs=[
            pl.BlockSpec((blk, 64), lambda i: (i, 0)),
            pl.BlockSpec((blk, 16), lambda i: (i, 0)),
            pl.BlockSpec((blk, 16), lambda i: (i, 0)),
        ],
        out_shape=[
            jax.ShapeDtypeStruct((N, 64), jnp.float32),
            jax.ShapeDtypeStruct((N, 16), jnp.float32),
            jax.ShapeDtypeStruct((N, 16), jnp.float32),
        ],
    )(x, W1, As1, Ad1)


def _edge1_body(ht, ast, adt, esrc, edst, zeros64, zeros16,
                acc_h_a, acc_h_b, acc_w_a, acc_w_b,
                sidx, didx, rows, asv, adv, con_h, con_w, acc_h, acc_w,
                sem_h, sem_s, sem_d):
    cid = lax.axis_index("c")
    sid = lax.axis_index("s")
    wid = cid * NS + sid

    pltpu.sync_copy(zeros64.at[pl.ds(sid * RPT, RPT)],
                    acc_h.at[pl.ds(sid * RPT, RPT)])
    pltpu.sync_copy(zeros16.at[pl.ds(sid * RPT, RPT)],
                    acc_w.at[pl.ds(sid * RPT, RPT)])
    plsc.subcore_barrier()

    lane = lax.iota(jnp.int32, 16)
    widx = lane >> 3

    def chunk(i, carry):
        base = wid * EPW + i * K
        pltpu.sync_copy(esrc.at[pl.ds(base, K)], sidx)
        pltpu.sync_copy(edst.at[pl.ds(base, K)], didx)
        ch = pltpu.async_copy(ht.at[sidx], rows, sem_h)
        cs = pltpu.async_copy(ast.at[sidx], asv, sem_s)
        cd = pltpu.async_copy(adt.at[didx], adv, sem_d)
        ch.wait()
        cs.wait()
        cd.wait()
        for e in range(K):
            s = asv[e, :] + adv[e, :]
            w16 = jnp.exp(jnp.maximum(s, 0.2 * s))
            con_w[e, :] = w16
            for j in range(4):
                hj = rows[e, pl.ds(16 * j, 16)]
                wj = _take16(w16, widx + 2 * j)
                con_h[e, pl.ds(16 * j, 16)] = hj * wj
        pltpu.sync_copy(con_h, acc_h.at[didx], add=True)
        pltpu.sync_copy(con_w, acc_w.at[didx], add=True)
        return carry

    lax.fori_loop(0, NCHUNK, chunk, 0)
    plsc.subcore_barrier()

    @pl.when(cid == 0)
    def _():
        pltpu.sync_copy(acc_h.at[pl.ds(sid * RPT, RPT)],
                        acc_h_a.at[pl.ds(sid * RPT, RPT)])
        pltpu.sync_copy(acc_w.at[pl.ds(sid * RPT, RPT)],
                        acc_w_a.at[pl.ds(sid * RPT, RPT)])

    @pl.when(cid == 1)
    def _():
        pltpu.sync_copy(acc_h.at[pl.ds(sid * RPT, RPT)],
                        acc_h_b.at[pl.ds(sid * RPT, RPT)])
        pltpu.sync_copy(acc_w.at[pl.ds(sid * RPT, RPT)],
                        acc_w_b.at[pl.ds(sid * RPT, RPT)])


def _edge1(ht, ast, adt, esrc, edst, zeros64, zeros16):
    mesh = plsc.VectorSubcoreMesh(core_axis_name="c", subcore_axis_name="s")
    fn = functools.partial(
        pl.kernel,
        out_type=[
            jax.ShapeDtypeStruct((NPAD, 64), jnp.float32),
            jax.ShapeDtypeStruct((NPAD, 64), jnp.float32),
            jax.ShapeDtypeStruct((NPAD, 16), jnp.float32),
            jax.ShapeDtypeStruct((NPAD, 16), jnp.float32),
        ],
        mesh=mesh,
        compiler_params=pltpu.CompilerParams(use_tc_tiling_on_sc=False),
        scratch_types=[
            pltpu.VMEM((K,), jnp.int32),
            pltpu.VMEM((K,), jnp.int32),
            pltpu.VMEM((K, 64), jnp.float32),
            pltpu.VMEM((K, 16), jnp.float32),
            pltpu.VMEM((K, 16), jnp.float32),
            pltpu.VMEM((K, 64), jnp.float32),
            pltpu.VMEM((K, 16), jnp.float32),
            pltpu.VMEM_SHARED((NPAD, 64), jnp.float32),
            pltpu.VMEM_SHARED((NPAD, 16), jnp.float32),
            pltpu.SemaphoreType.DMA,
            pltpu.SemaphoreType.DMA,
            pltpu.SemaphoreType.DMA,
        ],
    )(_edge1_body)
    return fn(ht, ast, adt, esrc, edst, zeros64, zeros16)


def _mid_body(ah_a_ref, ah_b_ref, aw_a_ref, aw_b_ref, ht_ref, ast_ref,
              adt_ref, b1_ref, w2_ref, as2_ref, ad2_ref, g2t_ref):
    blk = ht_ref.shape[0]
    h = ht_ref[...]
    als = ast_ref[:, :8]
    ald = adt_ref[:, :8]
    s = als + ald
    exs = jnp.exp(jnp.maximum(s, 0.2 * s))
    num = ah_a_ref[...] + ah_b_ref[...]
    num = num + (h.reshape(blk, H1, C1) * exs[:, :, None]).reshape(blk, 64)
    den = aw_a_ref[:, :8] + aw_b_ref[:, :8] + exs
    out1 = (num.reshape(blk, H1, C1) / den[:, :, None]).reshape(blk, 64)
    h2 = jnp.maximum(out1 + b1_ref[...], 0.0)
    g2 = jnp.dot(h2, w2_ref[...], preferred_element_type=jnp.float32)
    as2 = jnp.dot(g2, as2_ref[...].reshape(NUM_CLASSES, 1),
                  preferred_element_type=jnp.float32)
    ad2 = jnp.dot(g2, ad2_ref[...].reshape(NUM_CLASSES, 1),
                  preferred_element_type=jnp.float32)
    ones = jnp.ones((blk, 1), jnp.float32)
    pad = jnp.zeros((blk, 21), jnp.float32)
    g2t_ref[...] = jnp.concatenate([g2, as2, ones, ad2, pad], axis=1)


def _mid(ah_a, ah_b, aw_a, aw_b, ht, ast, adt, b1, W2, a_src2, a_dst2):
    blk = 1000
    return pl.pallas_call(
        _mid_body,
        grid=(N // blk,),
        in_specs=[
            pl.BlockSpec((blk, 64), lambda i: (i, 0)),
            pl.BlockSpec((blk, 64), lambda i: (i, 0)),
            pl.BlockSpec((blk, 16), lambda i: (i, 0)),
            pl.BlockSpec((blk, 16), lambda i: (i, 0)),
            pl.BlockSpec((blk, 64), lambda i: (i, 0)),
            pl.BlockSpec((blk, 16), lambda i: (i, 0)),
            pl.BlockSpec((blk, 16), lambda i: (i, 0)),
            pl.BlockSpec((1, 64), lambda i: (0, 0)),
            pl.BlockSpec((64, NUM_CLASSES), lambda i: (0, 0)),
            pl.BlockSpec((1, NUM_CLASSES), lambda i: (0, 0)),
            pl.BlockSpec((1, NUM_CLASSES), lambda i: (0, 0)),
        ],
        out_specs=pl.BlockSpec((blk, 64), lambda i: (i, 0)),
        out_shape=jax.ShapeDtypeStruct((N, 64), jnp.float32),
    )(ah_a, ah_b, aw_a, aw_b, ht, ast, adt, b1, W2, a_src2, a_dst2)


def _edge2_body(g2t, as2v, ad2v, esrc, edst, zeros64,
                acc_a, acc_b,
                sidx, didx, rows, asv, adv, wbuf, contrib, accum,
                sem_h, sem_a, sem_b):
    cid = lax.axis_index("c")
    sid = lax.axis_index("s")
    wid = cid * NS + sid

    pltpu.sync_copy(zeros64.at[pl.ds(sid * RPT, RPT)],
                    accum.at[pl.ds(sid * RPT, RPT)])
    plsc.subcore_barrier()

    lane = lax.iota(jnp.int32, 16)

    def chunk(i, carry):
        base = wid * EPW + i * K
        pltpu.sync_copy(esrc.at[pl.ds(base, K)], sidx)
        pltpu.sync_copy(edst.at[pl.ds(base, K)], didx)
        ch = pltpu.async_copy(g2t.at[sidx], rows, sem_h)
        cs = pltpu.async_copy(as2v.at[sidx], asv, sem_a)
        cd = pltpu.async_copy(ad2v.at[didx], adv, sem_b)
        ch.wait()
        cs.wait()
        cd.wait()
        for b in range(K // 16):
            s = asv[pl.ds(16 * b, 16)] + adv[pl.ds(16 * b, 16)]
            wbuf[pl.ds(16 * b, 16)] = jnp.exp(jnp.maximum(s, 0.2 * s))
        for g in range(K // 16):
            w16 = wbuf[pl.ds(16 * g, 16)]
            for r in range(16):
                e = 16 * g + r
                we = _take16(w16, lane * 0 + r)
                for j in range(4):
                    contrib[e, pl.ds(16 * j, 16)] = (
                        rows[e, pl.ds(16 * j, 16)] * we)
        pltpu.sync_copy(contrib, accum.at[didx], add=True)
        return carry

    lax.fori_loop(0, NCHUNK, chunk, 0)
    plsc.subcore_barrier()

    @pl.when(cid == 0)
    def _():
        pltpu.sync_copy(accum.at[pl.ds(sid * RPT, RPT)],
                        acc_a.at[pl.ds(sid * RPT, RPT)])

    @pl.when(cid == 1)
    def _():
        pltpu.sync_copy(accum.at[pl.ds(sid * RPT, RPT)],
                        acc_b.at[pl.ds(sid * RPT, RPT)])


def _edge2(g2t, as2v, ad2v, esrc, edst, zeros64):
    mesh = plsc.VectorSubcoreMesh(core_axis_name="c", subcore_axis_name="s")
    fn = functools.partial(
        pl.kernel,
        out_type=[
            jax.ShapeDtypeStruct((NPAD, 64), jnp.float32),
            jax.ShapeDtypeStruct((NPAD, 64), jnp.float32),
        ],
        mesh=mesh,
        compiler_params=pltpu.CompilerParams(use_tc_tiling_on_sc=False),
        scratch_types=[
            pltpu.VMEM((K,), jnp.int32),
            pltpu.VMEM((K,), jnp.int32),
            pltpu.VMEM((K, 64), jnp.float32),
            pltpu.VMEM((K,), jnp.float32),
            pltpu.VMEM((K,), jnp.float32),
            pltpu.VMEM((K,), jnp.float32),
            pltpu.VMEM((K, 64), jnp.float32),
            pltpu.VMEM_SHARED((NPAD, 64), jnp.float32),
            pltpu.SemaphoreType.DMA,
            pltpu.SemaphoreType.DMA,
            pltpu.SemaphoreType.DMA,
        ],
    )(_edge2_body)
    return fn(g2t, as2v, ad2v, esrc, edst, zeros64)


def _final_body(aa_ref, ab_ref, g2t_ref, b2_ref, out_ref):
    g2 = g2t_ref[:, :NUM_CLASSES]
    as2 = g2t_ref[:, 40:41]
    ad2 = g2t_ref[:, 42:43]
    s = as2 + ad2
    exs = jnp.exp(jnp.maximum(s, 0.2 * s))
    num = aa_ref[:, :NUM_CLASSES] + ab_ref[:, :NUM_CLASSES] + g2 * exs
    den = aa_ref[:, 41:42] + ab_ref[:, 41:42] + exs
    z = num / den + b2_ref[...]
    m = jnp.max(z, axis=1, keepdims=True)
    zs = z - m
    out_ref[...] = zs - jnp.log(jnp.sum(jnp.exp(zs), axis=1, keepdims=True))


def _final(acc_a, acc_b, g2t, b2):
    blk = 1000
    return pl.pallas_call(
        _final_body,
        grid=(N // blk,),
        in_specs=[
            pl.BlockSpec((blk, 64), lambda i: (i, 0)),
            pl.BlockSpec((blk, 64), lambda i: (i, 0)),
            pl.BlockSpec((blk, 64), lambda i: (i, 0)),
            pl.BlockSpec((1, NUM_CLASSES), lambda i: (0, 0)),
        ],
        out_specs=pl.BlockSpec((blk, NUM_CLASSES), lambda i: (i, 0)),
        out_shape=jax.ShapeDtypeStruct((N, NUM_CLASSES), jnp.float32),
    )(acc_a, acc_b, g2t, b2)


def kernel(x, edge_index, W1, a_src1, a_dst1, b1, W2, a_src2, a_dst2, b2):
    esrc = edge_index[0]
    edst = edge_index[1]

    eye = jnp.eye(H1, dtype=jnp.float32)
    As1 = (a_src1[:, :, None] * eye[:, None, :]).reshape(H1 * C1, H1)
    Ad1 = (a_dst1[:, :, None] * eye[:, None, :]).reshape(H1 * C1, H1)

    ht, ast, adt = _dense1(x, W1, As1, Ad1)
    zeros64 = jnp.zeros((NPAD, 64), jnp.float32)
    zeros16 = jnp.zeros((NPAD, 16), jnp.float32)
    ah_a, ah_b, aw_a, aw_b = _edge1(ht, ast, adt, esrc, edst,
                                    zeros64, zeros16)

    g2t = _mid(ah_a, ah_b, aw_a, aw_b, ht, ast, adt,
               b1.reshape(1, 64), W2, a_src2, a_dst2)
    as2v = g2t[:, 40]
    ad2v = g2t[:, 42]
    acc2_a, acc2_b = _edge2(g2t, as2v, ad2v, esrc, edst, zeros64)

    return _final(acc2_a, acc2_b, g2t, b2.reshape(1, NUM_CLASSES))


# speedup vs baseline: 131.1522x; 1.9076x over previous
import functools

import jax
import jax.numpy as jnp
from jax import lax
from jax.experimental import pallas as pl
from jax.experimental.pallas import tpu as pltpu
from jax.experimental.pallas import tpu_sc as plsc

N = 10000
E = 320000
F_IN = 128
H1, C1 = 8, 8
NUM_CLASSES = 40

NC, NS = 2, 16
NW = NC * NS
K = 80
EPW = E // NW
NCHUNK = EPW // K
NPAD = 10240
RPT = NPAD // NS


def _take16(vec, idx):
    return lax.gather(
        vec, idx[:, None],
        lax.GatherDimensionNumbers(offset_dims=(), collapsed_slice_dims=(0,),
                                   start_index_map=(0,)),
        slice_sizes=(1,),
        mode=lax.GatherScatterMode.PROMISE_IN_BOUNDS,
    )


def _dense1_body(x_ref, w_ref, as_ref, ad_ref, ht_ref, ast_ref, adt_ref):
    h = jnp.dot(x_ref[...], w_ref[...], preferred_element_type=jnp.float32)
    als = jnp.dot(h, as_ref[...], preferred_element_type=jnp.float32)
    ald = jnp.dot(h, ad_ref[...], preferred_element_type=jnp.float32)
    ht_ref[...] = h
    ast_ref[...] = jnp.concatenate([als, als], axis=1)
    adt_ref[...] = jnp.concatenate([ald, ald], axis=1)


def _dense1(x, W1, As1, Ad1):
    blk = 1000
    return pl.pallas_call(
        _dense1_body,
        grid=(N // blk,),
        in_specs=[
            pl.BlockSpec((blk, F_IN), lambda i: (i, 0)),
            pl.BlockSpec((F_IN, H1 * C1), lambda i: (0, 0)),
            pl.BlockSpec((H1 * C1, H1), lambda i: (0, 0)),
            pl.BlockSpec((H1 * C1, H1), lambda i: (0, 0)),
        ],
        out_specs=[
            pl.BlockSpec((blk, 64), lambda i: (i, 0)),
            pl.BlockSpec((blk, 16), lambda i: (i, 0)),
            pl.BlockSpec((blk, 16), lambda i: (i, 0)),
        ],
        out_shape=[
            jax.ShapeDtypeStruct((N, 64), jnp.float32),
            jax.ShapeDtypeStruct((N, 16), jnp.float32),
            jax.ShapeDtypeStruct((N, 16), jnp.float32),
        ],
    )(x, W1, As1, Ad1)


def _edge1_body(ht, ast, adt, esrc3, edst3, zeros64, zeros16,
                acc_h_a, acc_h_b, acc_w_a, acc_w_b,
                sia, dia, rows0, rows1, asv0, asv1, adv0, adv1,
                ch0, ch1, cw0, cw1, acc_h, acc_w,
                sh0, sh1, ss0, ss1, sd0, sd1):
    cid = lax.axis_index("c")
    sid = lax.axis_index("s")
    wid = cid * NS + sid
    rows, asv, adv = [rows0, rows1], [asv0, asv1], [adv0, adv1]
    con_h, con_w = [ch0, ch1], [cw0, cw1]
    sem_h, sem_s, sem_d = [sh0, sh1], [ss0, ss1], [sd0, sd1]

    pltpu.sync_copy(zeros64.at[pl.ds(sid * RPT, RPT)],
                    acc_h.at[pl.ds(sid * RPT, RPT)])
    pltpu.sync_copy(zeros16.at[pl.ds(sid * RPT, RPT)],
                    acc_w.at[pl.ds(sid * RPT, RPT)])
    pltpu.sync_copy(esrc3.at[wid], sia)
    pltpu.sync_copy(edst3.at[wid], dia)

    def fire(i, b):
        pltpu.async_copy(ht.at[sia.at[i]], rows[b], sem_h[b])
        pltpu.async_copy(ast.at[sia.at[i]], asv[b], sem_s[b])
        pltpu.async_copy(adt.at[dia.at[i]], adv[b], sem_d[b])

    def wait(i, b):
        pltpu.make_async_copy(ht.at[sia.at[i]], rows[b], sem_h[b]).wait()
        pltpu.make_async_copy(ast.at[sia.at[i]], asv[b], sem_s[b]).wait()
        pltpu.make_async_copy(adt.at[dia.at[i]], adv[b], sem_d[b]).wait()

    fire(0, 0)
    fire(1, 1)
    plsc.subcore_barrier()

    lane = lax.iota(jnp.int32, 16)
    widx = lane >> 3

    def step(i, b):
        wait(i, b)
        for e in range(K):
            s = asv[b][e, :] + adv[b][e, :]
            w16 = jnp.exp(jnp.maximum(s, 0.2 * s))
            con_w[b][e, :] = w16
            for j in range(4):
                hj = rows[b][e, pl.ds(16 * j, 16)]
                wj = _take16(w16, widx + 2 * j)
                con_h[b][e, pl.ds(16 * j, 16)] = hj * wj
        pltpu.sync_copy(con_h[b], acc_h.at[dia.at[i]], add=True)
        pltpu.sync_copy(con_w[b], acc_w.at[dia.at[i]], add=True)

        @pl.when(i + 2 < NCHUNK)
        def _():
            fire(i + 2, b)

    def pair(it, carry):
        step(2 * it, 0)
        step(2 * it + 1, 1)
        return carry

    lax.fori_loop(0, NCHUNK // 2, pair, 0)
    if NCHUNK % 2:
        step(NCHUNK - 1, 0)
    plsc.subcore_barrier()

    @pl.when(cid == 0)
    def _():
        pltpu.sync_copy(acc_h.at[pl.ds(sid * RPT, RPT)],
                        acc_h_a.at[pl.ds(sid * RPT, RPT)])
        pltpu.sync_copy(acc_w.at[pl.ds(sid * RPT, RPT)],
                        acc_w_a.at[pl.ds(sid * RPT, RPT)])

    @pl.when(cid == 1)
    def _():
        pltpu.sync_copy(acc_h.at[pl.ds(sid * RPT, RPT)],
                        acc_h_b.at[pl.ds(sid * RPT, RPT)])
        pltpu.sync_copy(acc_w.at[pl.ds(sid * RPT, RPT)],
                        acc_w_b.at[pl.ds(sid * RPT, RPT)])


def _edge1(ht, ast, adt, esrc3, edst3, zeros64, zeros16):
    mesh = plsc.VectorSubcoreMesh(core_axis_name="c", subcore_axis_name="s")
    fn = functools.partial(
        pl.kernel,
        out_type=[
            jax.ShapeDtypeStruct((NPAD, 64), jnp.float32),
            jax.ShapeDtypeStruct((NPAD, 64), jnp.float32),
            jax.ShapeDtypeStruct((NPAD, 16), jnp.float32),
            jax.ShapeDtypeStruct((NPAD, 16), jnp.float32),
        ],
        mesh=mesh,
        compiler_params=pltpu.CompilerParams(use_tc_tiling_on_sc=False),
        scratch_types=[
            pltpu.VMEM((NCHUNK, K), jnp.int32),
            pltpu.VMEM((NCHUNK, K), jnp.int32),
            pltpu.VMEM((K, 64), jnp.float32),
            pltpu.VMEM((K, 64), jnp.float32),
            pltpu.VMEM((K, 16), jnp.float32),
            pltpu.VMEM((K, 16), jnp.float32),
            pltpu.VMEM((K, 16), jnp.float32),
            pltpu.VMEM((K, 16), jnp.float32),
            pltpu.VMEM((K, 64), jnp.float32),
            pltpu.VMEM((K, 64), jnp.float32),
            pltpu.VMEM((K, 16), jnp.float32),
            pltpu.VMEM((K, 16), jnp.float32),
            pltpu.VMEM_SHARED((NPAD, 64), jnp.float32),
            pltpu.VMEM_SHARED((NPAD, 16), jnp.float32),
            pltpu.SemaphoreType.DMA,
            pltpu.SemaphoreType.DMA,
            pltpu.SemaphoreType.DMA,
            pltpu.SemaphoreType.DMA,
            pltpu.SemaphoreType.DMA,
            pltpu.SemaphoreType.DMA,
        ],
    )(_edge1_body)
    return fn(ht, ast, adt, esrc3, edst3, zeros64, zeros16)


def _mid_body(ah_a_ref, ah_b_ref, aw_a_ref, aw_b_ref, ht_ref, ast_ref,
              adt_ref, b1_ref, w2_ref, as2_ref, ad2_ref, g2t_ref):
    blk = ht_ref.shape[0]
    h = ht_ref[...]
    als = ast_ref[:, :8]
    ald = adt_ref[:, :8]
    s = als + ald
    exs = jnp.exp(jnp.maximum(s, 0.2 * s))
    num = ah_a_ref[...] + ah_b_ref[...]
    num = num + (h.reshape(blk, H1, C1) * exs[:, :, None]).reshape(blk, 64)
    den = aw_a_ref[:, :8] + aw_b_ref[:, :8] + exs
    out1 = (num.reshape(blk, H1, C1) / den[:, :, None]).reshape(blk, 64)
    h2 = jnp.maximum(out1 + b1_ref[...], 0.0)
    g2 = jnp.dot(h2, w2_ref[...], preferred_element_type=jnp.float32)
    as2 = jnp.dot(g2, as2_ref[...].reshape(NUM_CLASSES, 1),
                  preferred_element_type=jnp.float32)
    ad2 = jnp.dot(g2, ad2_ref[...].reshape(NUM_CLASSES, 1),
                  preferred_element_type=jnp.float32)
    ones = jnp.ones((blk, 1), jnp.float32)
    pad = jnp.zeros((blk, 21), jnp.float32)
    g2t_ref[...] = jnp.concatenate([g2, as2, ones, ad2, pad], axis=1)


def _mid(ah_a, ah_b, aw_a, aw_b, ht, ast, adt, b1, W2, a_src2, a_dst2):
    blk = 1000
    return pl.pallas_call(
        _mid_body,
        grid=(N // blk,),
        in_specs=[
            pl.BlockSpec((blk, 64), lambda i: (i, 0)),
            pl.BlockSpec((blk, 64), lambda i: (i, 0)),
            pl.BlockSpec((blk, 16), lambda i: (i, 0)),
            pl.BlockSpec((blk, 16), lambda i: (i, 0)),
            pl.BlockSpec((blk, 64), lambda i: (i, 0)),
            pl.BlockSpec((blk, 16), lambda i: (i, 0)),
            pl.BlockSpec((blk, 16), lambda i: (i, 0)),
            pl.BlockSpec((1, 64), lambda i: (0, 0)),
            pl.BlockSpec((64, NUM_CLASSES), lambda i: (0, 0)),
            pl.BlockSpec((1, NUM_CLASSES), lambda i: (0, 0)),
            pl.BlockSpec((1, NUM_CLASSES), lambda i: (0, 0)),
        ],
        out_specs=pl.BlockSpec((blk, 64), lambda i: (i, 0)),
        out_shape=jax.ShapeDtypeStruct((N, 64), jnp.float32),
    )(ah_a, ah_b, aw_a, aw_b, ht, ast, adt, b1, W2, a_src2, a_dst2)


def _edge2_body(g2t, as2v, ad2v, esrc3, edst3, zeros64,
                acc_a, acc_b,
                sia, dia, rows0, rows1, asv0, asv1, adv0, adv1,
                wbuf, co0, co1, accum,
                sh0, sh1, ss0, ss1, sd0, sd1):
    cid = lax.axis_index("c")
    sid = lax.axis_index("s")
    wid = cid * NS + sid
    rows, asv, adv = [rows0, rows1], [asv0, asv1], [adv0, adv1]
    contrib = [co0, co1]
    sem_h, sem_s, sem_d = [sh0, sh1], [ss0, ss1], [sd0, sd1]

    pltpu.sync_copy(zeros64.at[pl.ds(sid * RPT, RPT)],
                    accum.at[pl.ds(sid * RPT, RPT)])
    pltpu.sync_copy(esrc3.at[wid], sia)
    pltpu.sync_copy(edst3.at[wid], dia)

    def fire(i, b):
        pltpu.async_copy(g2t.at[sia.at[i]], rows[b], sem_h[b])
        pltpu.async_copy(as2v.at[sia.at[i]], asv[b], sem_s[b])
        pltpu.async_copy(ad2v.at[dia.at[i]], adv[b], sem_d[b])

    def wait(i, b):
        pltpu.make_async_copy(g2t.at[sia.at[i]], rows[b], sem_h[b]).wait()
        pltpu.make_async_copy(as2v.at[sia.at[i]], asv[b], sem_s[b]).wait()
        pltpu.make_async_copy(ad2v.at[dia.at[i]], adv[b], sem_d[b]).wait()

    fire(0, 0)
    fire(1, 1)
    plsc.subcore_barrier()

    lane = lax.iota(jnp.int32, 16)

    def step(i, b):
        wait(i, b)
        for blk in range(K // 16):
            sv = asv[b][pl.ds(16 * blk, 16)] + adv[b][pl.ds(16 * blk, 16)]
            wbuf[pl.ds(16 * blk, 16)] = jnp.exp(jnp.maximum(sv, 0.2 * sv))
        for g in range(K // 16):
            w16 = wbuf[pl.ds(16 * g, 16)]
            for r in range(16):
                e = 16 * g + r
                we = _take16(w16, lane * 0 + r)
                for j in range(4):
                    contrib[b][e, pl.ds(16 * j, 16)] = (
                        rows[b][e, pl.ds(16 * j, 16)] * we)
        pltpu.sync_copy(contrib[b], accum.at[dia.at[i]], add=True)

        @pl.when(i + 2 < NCHUNK)
        def _():
            fire(i + 2, b)

    def pair(it, carry):
        step(2 * it, 0)
        step(2 * it + 1, 1)
        return carry

    lax.fori_loop(0, NCHUNK // 2, pair, 0)
    if NCHUNK % 2:
        step(NCHUNK - 1, 0)
    plsc.subcore_barrier()

    @pl.when(cid == 0)
    def _():
        pltpu.sync_copy(accum.at[pl.ds(sid * RPT, RPT)],
                        acc_a.at[pl.ds(sid * RPT, RPT)])

    @pl.when(cid == 1)
    def _():
        pltpu.sync_copy(accum.at[pl.ds(sid * RPT, RPT)],
                        acc_b.at[pl.ds(sid * RPT, RPT)])


def _edge2(g2t, as2v, ad2v, esrc3, edst3, zeros64):
    mesh = plsc.VectorSubcoreMesh(core_axis_name="c", subcore_axis_name="s")
    fn = functools.partial(
        pl.kernel,
        out_type=[
            jax.ShapeDtypeStruct((NPAD, 64), jnp.float32),
            jax.ShapeDtypeStruct((NPAD, 64), jnp.float32),
        ],
        mesh=mesh,
        compiler_params=pltpu.CompilerParams(use_tc_tiling_on_sc=False),
        scratch_types=[
            pltpu.VMEM((NCHUNK, K), jnp.int32),
            pltpu.VMEM((NCHUNK, K), jnp.int32),
            pltpu.VMEM((K, 64), jnp.float32),
            pltpu.VMEM((K, 64), jnp.float32),
            pltpu.VMEM((K,), jnp.float32),
            pltpu.VMEM((K,), jnp.float32),
            pltpu.VMEM((K,), jnp.float32),
            pltpu.VMEM((K,), jnp.float32),
            pltpu.VMEM((K,), jnp.float32),
            pltpu.VMEM((K, 64), jnp.float32),
            pltpu.VMEM((K, 64), jnp.float32),
            pltpu.VMEM_SHARED((NPAD, 64), jnp.float32),
            pltpu.SemaphoreType.DMA,
            pltpu.SemaphoreType.DMA,
            pltpu.SemaphoreType.DMA,
            pltpu.SemaphoreType.DMA,
            pltpu.SemaphoreType.DMA,
            pltpu.SemaphoreType.DMA,
        ],
    )(_edge2_body)
    return fn(g2t, as2v, ad2v, esrc3, edst3, zeros64)


def _final_body(aa_ref, ab_ref, g2t_ref, b2_ref, out_ref):
    g2 = g2t_ref[:, :NUM_CLASSES]
    as2 = g2t_ref[:, 40:41]
    ad2 = g2t_ref[:, 42:43]
    s = as2 + ad2
    exs = jnp.exp(jnp.maximum(s, 0.2 * s))
    num = aa_ref[:, :NUM_CLASSES] + ab_ref[:, :NUM_CLASSES] + g2 * exs
    den = aa_ref[:, 41:42] + ab_ref[:, 41:42] + exs
    z = num / den + b2_ref[...]
    m = jnp.max(z, axis=1, keepdims=True)
    zs = z - m
    out_ref[...] = zs - jnp.log(jnp.sum(jnp.exp(zs), axis=1, keepdims=True))


def _final(acc_a, acc_b, g2t, b2):
    blk = 1000
    return pl.pallas_call(
        _final_body,
        grid=(N // blk,),
        in_specs=[
            pl.BlockSpec((blk, 64), lambda i: (i, 0)),
            pl.BlockSpec((blk, 64), lambda i: (i, 0)),
            pl.BlockSpec((blk, 64), lambda i: (i, 0)),
            pl.BlockSpec((1, NUM_CLASSES), lambda i: (0, 0)),
        ],
        out_specs=pl.BlockSpec((blk, NUM_CLASSES), lambda i: (i, 0)),
        out_shape=jax.ShapeDtypeStruct((N, NUM_CLASSES), jnp.float32),
    )(acc_a, acc_b, g2t, b2)


def kernel(x, edge_index, W1, a_src1, a_dst1, b1, W2, a_src2, a_dst2, b2):
    esrc3 = edge_index[0].reshape(NW, NCHUNK, K)
    edst3 = edge_index[1].reshape(NW, NCHUNK, K)

    eye = jnp.eye(H1, dtype=jnp.float32)
    As1 = (a_src1[:, :, None] * eye[:, None, :]).reshape(H1 * C1, H1)
    Ad1 = (a_dst1[:, :, None] * eye[:, None, :]).reshape(H1 * C1, H1)

    ht, ast, adt = _dense1(x, W1, As1, Ad1)
    zeros64 = jnp.zeros((NPAD, 64), jnp.float32)
    zeros16 = jnp.zeros((NPAD, 16), jnp.float32)
    ah_a, ah_b, aw_a, aw_b = _edge1(ht, ast, adt, esrc3, edst3,
                                    zeros64, zeros16)

    g2t = _mid(ah_a, ah_b, aw_a, aw_b, ht, ast, adt,
               b1.reshape(1, 64), W2, a_src2, a_dst2)
    as2v = g2t[:, 40]
    ad2v = g2t[:, 42]
    acc2_a, acc2_b = _edge2(g2t, as2v, ad2v, esrc3, edst3, zeros64)

    return _final(acc2_a, acc2_b, g2t, b2.reshape(1, NUM_CLASSES))


# speedup vs baseline: 143.8023x; 1.0965x over previous
import functools

import jax
import jax.numpy as jnp
from jax import lax
from jax.experimental import pallas as pl
from jax.experimental.pallas import tpu as pltpu
from jax.experimental.pallas import tpu_sc as plsc

N = 10000
E = 320000
F_IN = 128
H1, C1 = 8, 8
NUM_CLASSES = 40

NC, NS = 2, 16
NW = NC * NS
K = 100
EPW = E // NW
NCHUNK = EPW // K
WSTARTS = [16 * b for b in range(K // 16)] + ([K - 16] if K % 16 else [])
NPAD = 10240
RPT = NPAD // NS


def _take16(vec, idx):
    return lax.gather(
        vec, idx[:, None],
        lax.GatherDimensionNumbers(offset_dims=(), collapsed_slice_dims=(0,),
                                   start_index_map=(0,)),
        slice_sizes=(1,),
        mode=lax.GatherScatterMode.PROMISE_IN_BOUNDS,
    )


def _dense1_body(x_ref, w_ref, as_ref, ad_ref, ht_ref, ast_ref, adt_ref):
    h = jnp.dot(x_ref[...], w_ref[...], preferred_element_type=jnp.float32)
    als = jnp.dot(h, as_ref[...], preferred_element_type=jnp.float32)
    ald = jnp.dot(h, ad_ref[...], preferred_element_type=jnp.float32)
    ht_ref[...] = h
    ast_ref[...] = jnp.concatenate([als, als], axis=1)
    adt_ref[...] = jnp.concatenate([ald, ald], axis=1)


def _dense1(x, W1, As1, Ad1):
    blk = 1000
    return pl.pallas_call(
        _dense1_body,
        grid=(N // blk,),
        in_specs=[
            pl.BlockSpec((blk, F_IN), lambda i: (i, 0)),
            pl.BlockSpec((F_IN, H1 * C1), lambda i: (0, 0)),
            pl.BlockSpec((H1 * C1, H1), lambda i: (0, 0)),
            pl.BlockSpec((H1 * C1, H1), lambda i: (0, 0)),
        ],
        out_specs=[
            pl.BlockSpec((blk, 64), lambda i: (i, 0)),
            pl.BlockSpec((blk, 16), lambda i: (i, 0)),
            pl.BlockSpec((blk, 16), lambda i: (i, 0)),
        ],
        out_shape=[
            jax.ShapeDtypeStruct((N, 64), jnp.float32),
            jax.ShapeDtypeStruct((N, 16), jnp.float32),
            jax.ShapeDtypeStruct((N, 16), jnp.float32),
        ],
    )(x, W1, As1, Ad1)


def _edge1_body(ht, ast, adt, esrc3, edst3, zeros64, zeros16,
                acc_h_a, acc_h_b, acc_w_a, acc_w_b,
                sia, dia, rows0, rows1, asv0, asv1, adv0, adv1,
                ch0, ch1, cw0, cw1, acc_h, acc_w,
                sh0, sh1, ss0, ss1, sd0, sd1, qh0, qh1, qw0, qw1):
    cid = lax.axis_index("c")
    sid = lax.axis_index("s")
    wid = cid * NS + sid
    rows, asv, adv = [rows0, rows1], [asv0, asv1], [adv0, adv1]
    con_h, con_w = [ch0, ch1], [cw0, cw1]
    sem_h, sem_s, sem_d = [sh0, sh1], [ss0, ss1], [sd0, sd1]
    sem_qh, sem_qw = [qh0, qh1], [qw0, qw1]

    pltpu.sync_copy(zeros64.at[pl.ds(sid * RPT, RPT)],
                    acc_h.at[pl.ds(sid * RPT, RPT)])
    pltpu.sync_copy(zeros16.at[pl.ds(sid * RPT, RPT)],
                    acc_w.at[pl.ds(sid * RPT, RPT)])
    pltpu.sync_copy(esrc3.at[wid], sia)
    pltpu.sync_copy(edst3.at[wid], dia)

    def fire(i, b):
        pltpu.async_copy(ht.at[sia.at[i]], rows[b], sem_h[b])
        pltpu.async_copy(ast.at[sia.at[i]], asv[b], sem_s[b])
        pltpu.async_copy(adt.at[dia.at[i]], adv[b], sem_d[b])

    def wait(i, b):
        pltpu.make_async_copy(ht.at[sia.at[i]], rows[b], sem_h[b]).wait()
        pltpu.make_async_copy(ast.at[sia.at[i]], asv[b], sem_s[b]).wait()
        pltpu.make_async_copy(adt.at[dia.at[i]], adv[b], sem_d[b]).wait()

    fire(0, 0)
    fire(1, 1)
    plsc.subcore_barrier()

    lane = lax.iota(jnp.int32, 16)
    widx = lane >> 3

    def step(i, b):
        @pl.when(i >= 2)
        def _():
            pltpu.make_async_copy(con_h[b], acc_h.at[dia.at[i - 2]],
                                  sem_qh[b]).wait()
            pltpu.make_async_copy(con_w[b], acc_w.at[dia.at[i - 2]],
                                  sem_qw[b]).wait()
        wait(i, b)
        for e in range(K):
            s = asv[b][e, :] + adv[b][e, :]
            w16 = jnp.exp(jnp.maximum(s, 0.2 * s))
            con_w[b][e, :] = w16
            for j in range(4):
                hj = rows[b][e, pl.ds(16 * j, 16)]
                wj = _take16(w16, widx + 2 * j)
                con_h[b][e, pl.ds(16 * j, 16)] = hj * wj
        pltpu.async_copy(con_h[b], acc_h.at[dia.at[i]], sem_qh[b], add=True)
        pltpu.async_copy(con_w[b], acc_w.at[dia.at[i]], sem_qw[b], add=True)

        @pl.when(i + 2 < NCHUNK)
        def _():
            fire(i + 2, b)

    def pair(it, carry):
        step(2 * it, 0)
        step(2 * it + 1, 1)
        return carry

    lax.fori_loop(0, NCHUNK // 2, pair, 0)
    for b in range(2):
        i = NCHUNK - 2 + b
        pltpu.make_async_copy(con_h[b], acc_h.at[dia.at[i]],
                              sem_qh[b]).wait()
        pltpu.make_async_copy(con_w[b], acc_w.at[dia.at[i]],
                              sem_qw[b]).wait()
    plsc.subcore_barrier()

    @pl.when(cid == 0)
    def _():
        pltpu.sync_copy(acc_h.at[pl.ds(sid * RPT, RPT)],
                        acc_h_a.at[pl.ds(sid * RPT, RPT)])
        pltpu.sync_copy(acc_w.at[pl.ds(sid * RPT, RPT)],
                        acc_w_a.at[pl.ds(sid * RPT, RPT)])

    @pl.when(cid == 1)
    def _():
        pltpu.sync_copy(acc_h.at[pl.ds(sid * RPT, RPT)],
                        acc_h_b.at[pl.ds(sid * RPT, RPT)])
        pltpu.sync_copy(acc_w.at[pl.ds(sid * RPT, RPT)],
                        acc_w_b.at[pl.ds(sid * RPT, RPT)])


def _edge1(ht, ast, adt, esrc3, edst3, zeros64, zeros16):
    mesh = plsc.VectorSubcoreMesh(core_axis_name="c", subcore_axis_name="s")
    fn = functools.partial(
        pl.kernel,
        out_type=[
            jax.ShapeDtypeStruct((NPAD, 64), jnp.float32),
            jax.ShapeDtypeStruct((NPAD, 64), jnp.float32),
            jax.ShapeDtypeStruct((NPAD, 16), jnp.float32),
            jax.ShapeDtypeStruct((NPAD, 16), jnp.float32),
        ],
        mesh=mesh,
        compiler_params=pltpu.CompilerParams(use_tc_tiling_on_sc=False),
        scratch_types=[
            pltpu.VMEM((NCHUNK, K), jnp.int32),
            pltpu.VMEM((NCHUNK, K), jnp.int32),
            pltpu.VMEM((K, 64), jnp.float32),
            pltpu.VMEM((K, 64), jnp.float32),
            pltpu.VMEM((K, 16), jnp.float32),
            pltpu.VMEM((K, 16), jnp.float32),
            pltpu.VMEM((K, 16), jnp.float32),
            pltpu.VMEM((K, 16), jnp.float32),
            pltpu.VMEM((K, 64), jnp.float32),
            pltpu.VMEM((K, 64), jnp.float32),
            pltpu.VMEM((K, 16), jnp.float32),
            pltpu.VMEM((K, 16), jnp.float32),
            pltpu.VMEM_SHARED((NPAD, 64), jnp.float32),
            pltpu.VMEM_SHARED((NPAD, 16), jnp.float32),
            pltpu.SemaphoreType.DMA,
            pltpu.SemaphoreType.DMA,
            pltpu.SemaphoreType.DMA,
            pltpu.SemaphoreType.DMA,
            pltpu.SemaphoreType.DMA,
            pltpu.SemaphoreType.DMA,
            pltpu.SemaphoreType.DMA,
            pltpu.SemaphoreType.DMA,
            pltpu.SemaphoreType.DMA,
            pltpu.SemaphoreType.DMA,
        ],
    )(_edge1_body)
    return fn(ht, ast, adt, esrc3, edst3, zeros64, zeros16)


def _mid_body(ah_a_ref, ah_b_ref, aw_a_ref, aw_b_ref, ht_ref, ast_ref,
              adt_ref, b1_ref, w2_ref, as2_ref, ad2_ref, g2t_ref):
    blk = ht_ref.shape[0]
    h = ht_ref[...]
    als = ast_ref[:, :8]
    ald = adt_ref[:, :8]
    s = als + ald
    exs = jnp.exp(jnp.maximum(s, 0.2 * s))
    num = ah_a_ref[...] + ah_b_ref[...]
    num = num + (h.reshape(blk, H1, C1) * exs[:, :, None]).reshape(blk, 64)
    den = aw_a_ref[:, :8] + aw_b_ref[:, :8] + exs
    out1 = (num.reshape(blk, H1, C1) / den[:, :, None]).reshape(blk, 64)
    h2 = jnp.maximum(out1 + b1_ref[...], 0.0)
    g2 = jnp.dot(h2, w2_ref[...], preferred_element_type=jnp.float32)
    as2 = jnp.dot(g2, as2_ref[...].reshape(NUM_CLASSES, 1),
                  preferred_element_type=jnp.float32)
    ad2 = jnp.dot(g2, ad2_ref[...].reshape(NUM_CLASSES, 1),
                  preferred_element_type=jnp.float32)
    ones = jnp.ones((blk, 1), jnp.float32)
    pad = jnp.zeros((blk, 21), jnp.float32)
    g2t_ref[...] = jnp.concatenate([g2, as2, ones, ad2, pad], axis=1)


def _mid(ah_a, ah_b, aw_a, aw_b, ht, ast, adt, b1, W2, a_src2, a_dst2):
    blk = 1000
    return pl.pallas_call(
        _mid_body,
        grid=(N // blk,),
        in_specs=[
            pl.BlockSpec((blk, 64), lambda i: (i, 0)),
            pl.BlockSpec((blk, 64), lambda i: (i, 0)),
            pl.BlockSpec((blk, 16), lambda i: (i, 0)),
            pl.BlockSpec((blk, 16), lambda i: (i, 0)),
            pl.BlockSpec((blk, 64), lambda i: (i, 0)),
            pl.BlockSpec((blk, 16), lambda i: (i, 0)),
            pl.BlockSpec((blk, 16), lambda i: (i, 0)),
            pl.BlockSpec((1, 64), lambda i: (0, 0)),
            pl.BlockSpec((64, NUM_CLASSES), lambda i: (0, 0)),
            pl.BlockSpec((1, NUM_CLASSES), lambda i: (0, 0)),
            pl.BlockSpec((1, NUM_CLASSES), lambda i: (0, 0)),
        ],
        out_specs=pl.BlockSpec((blk, 64), lambda i: (i, 0)),
        out_shape=jax.ShapeDtypeStruct((N, 64), jnp.float32),
    )(ah_a, ah_b, aw_a, aw_b, ht, ast, adt, b1, W2, a_src2, a_dst2)


def _edge2_body(g2t, as2v, ad2v, esrc3, edst3, zeros64,
                acc_a, acc_b,
                sia, dia, rows0, rows1, asv0, asv1, adv0, adv1,
                wbuf, co0, co1, accum,
                sh0, sh1, ss0, ss1, sd0, sd1, q0, q1):
    cid = lax.axis_index("c")
    sid = lax.axis_index("s")
    wid = cid * NS + sid
    rows, asv, adv = [rows0, rows1], [asv0, asv1], [adv0, adv1]
    contrib = [co0, co1]
    sem_h, sem_s, sem_d = [sh0, sh1], [ss0, ss1], [sd0, sd1]
    sem_q = [q0, q1]

    pltpu.sync_copy(zeros64.at[pl.ds(sid * RPT, RPT)],
                    accum.at[pl.ds(sid * RPT, RPT)])
    pltpu.sync_copy(esrc3.at[wid], sia)
    pltpu.sync_copy(edst3.at[wid], dia)

    def fire(i, b):
        pltpu.async_copy(g2t.at[sia.at[i]], rows[b], sem_h[b])
        pltpu.async_copy(as2v.at[sia.at[i]], asv[b], sem_s[b])
        pltpu.async_copy(ad2v.at[dia.at[i]], adv[b], sem_d[b])

    def wait(i, b):
        pltpu.make_async_copy(g2t.at[sia.at[i]], rows[b], sem_h[b]).wait()
        pltpu.make_async_copy(as2v.at[sia.at[i]], asv[b], sem_s[b]).wait()
        pltpu.make_async_copy(ad2v.at[dia.at[i]], adv[b], sem_d[b]).wait()

    fire(0, 0)
    fire(1, 1)
    plsc.subcore_barrier()

    lane = lax.iota(jnp.int32, 16)

    def step(i, b):
        @pl.when(i >= 2)
        def _():
            pltpu.make_async_copy(contrib[b], accum.at[dia.at[i - 2]],
                                  sem_q[b]).wait()
        wait(i, b)
        for st in WSTARTS:
            sv = asv[b][pl.ds(st, 16)] + adv[b][pl.ds(st, 16)]
            wbuf[pl.ds(st, 16)] = jnp.exp(jnp.maximum(sv, 0.2 * sv))
        for st in WSTARTS:
            w16 = wbuf[pl.ds(st, 16)]
            hi = min(st + 16, K)
            for e in range(st, hi):
                we = _take16(w16, lane * 0 + (e - st))
                for j in range(4):
                    contrib[b][e, pl.ds(16 * j, 16)] = (
                        rows[b][e, pl.ds(16 * j, 16)] * we)
        pltpu.async_copy(contrib[b], accum.at[dia.at[i]], sem_q[b],
                         add=True)

        @pl.when(i + 2 < NCHUNK)
        def _():
            fire(i + 2, b)

    def pair(it, carry):
        step(2 * it, 0)
        step(2 * it + 1, 1)
        return carry

    lax.fori_loop(0, NCHUNK // 2, pair, 0)
    for b in range(2):
        pltpu.make_async_copy(contrib[b], accum.at[dia.at[NCHUNK - 2 + b]],
                              sem_q[b]).wait()
    plsc.subcore_barrier()

    @pl.when(cid == 0)
    def _():
        pltpu.sync_copy(accum.at[pl.ds(sid * RPT, RPT)],
                        acc_a.at[pl.ds(sid * RPT, RPT)])

    @pl.when(cid == 1)
    def _():
        pltpu.sync_copy(accum.at[pl.ds(sid * RPT, RPT)],
                        acc_b.at[pl.ds(sid * RPT, RPT)])


def _edge2(g2t, as2v, ad2v, esrc3, edst3, zeros64):
    mesh = plsc.VectorSubcoreMesh(core_axis_name="c", subcore_axis_name="s")
    fn = functools.partial(
        pl.kernel,
        out_type=[
            jax.ShapeDtypeStruct((NPAD, 64), jnp.float32),
            jax.ShapeDtypeStruct((NPAD, 64), jnp.float32),
        ],
        mesh=mesh,
        compiler_params=pltpu.CompilerParams(use_tc_tiling_on_sc=False),
        scratch_types=[
            pltpu.VMEM((NCHUNK, K), jnp.int32),
            pltpu.VMEM((NCHUNK, K), jnp.int32),
            pltpu.VMEM((K, 64), jnp.float32),
            pltpu.VMEM((K, 64), jnp.float32),
            pltpu.VMEM((K,), jnp.float32),
            pltpu.VMEM((K,), jnp.float32),
            pltpu.VMEM((K,), jnp.float32),
            pltpu.VMEM((K,), jnp.float32),
            pltpu.VMEM((K,), jnp.float32),
            pltpu.VMEM((K, 64), jnp.float32),
            pltpu.VMEM((K, 64), jnp.float32),
            pltpu.VMEM_SHARED((NPAD, 64), jnp.float32),
            pltpu.SemaphoreType.DMA,
            pltpu.SemaphoreType.DMA,
            pltpu.SemaphoreType.DMA,
            pltpu.SemaphoreType.DMA,
            pltpu.SemaphoreType.DMA,
            pltpu.SemaphoreType.DMA,
            pltpu.SemaphoreType.DMA,
            pltpu.SemaphoreType.DMA,
        ],
    )(_edge2_body)
    return fn(g2t, as2v, ad2v, esrc3, edst3, zeros64)


def _final_body(aa_ref, ab_ref, g2t_ref, b2_ref, out_ref):
    g2 = g2t_ref[:, :NUM_CLASSES]
    as2 = g2t_ref[:, 40:41]
    ad2 = g2t_ref[:, 42:43]
    s = as2 + ad2
    exs = jnp.exp(jnp.maximum(s, 0.2 * s))
    num = aa_ref[:, :NUM_CLASSES] + ab_ref[:, :NUM_CLASSES] + g2 * exs
    den = aa_ref[:, 41:42] + ab_ref[:, 41:42] + exs
    z = num / den + b2_ref[...]
    m = jnp.max(z, axis=1, keepdims=True)
    zs = z - m
    out_ref[...] = zs - jnp.log(jnp.sum(jnp.exp(zs), axis=1, keepdims=True))


def _final(acc_a, acc_b, g2t, b2):
    blk = 1000
    return pl.pallas_call(
        _final_body,
        grid=(N // blk,),
        in_specs=[
            pl.BlockSpec((blk, 64), lambda i: (i, 0)),
            pl.BlockSpec((blk, 64), lambda i: (i, 0)),
            pl.BlockSpec((blk, 64), lambda i: (i, 0)),
            pl.BlockSpec((1, NUM_CLASSES), lambda i: (0, 0)),
        ],
        out_specs=pl.BlockSpec((blk, NUM_CLASSES), lambda i: (i, 0)),
        out_shape=jax.ShapeDtypeStruct((N, NUM_CLASSES), jnp.float32),
    )(acc_a, acc_b, g2t, b2)


def kernel(x, edge_index, W1, a_src1, a_dst1, b1, W2, a_src2, a_dst2, b2):
    esrc3 = edge_index[0].reshape(NW, NCHUNK, K)
    edst3 = edge_index[1].reshape(NW, NCHUNK, K)

    eye = jnp.eye(H1, dtype=jnp.float32)
    As1 = (a_src1[:, :, None] * eye[:, None, :]).reshape(H1 * C1, H1)
    Ad1 = (a_dst1[:, :, None] * eye[:, None, :]).reshape(H1 * C1, H1)

    ht, ast, adt = _dense1(x, W1, As1, Ad1)
    zeros64 = jnp.zeros((NPAD, 64), jnp.float32)
    zeros16 = jnp.zeros((NPAD, 16), jnp.float32)
    ah_a, ah_b, aw_a, aw_b = _edge1(ht, ast, adt, esrc3, edst3,
                                    zeros64, zeros16)

    g2t = _mid(ah_a, ah_b, aw_a, aw_b, ht, ast, adt,
               b1.reshape(1, 64), W2, a_src2, a_dst2)
    as2v = g2t[:, 40]
    ad2v = g2t[:, 42]
    acc2_a, acc2_b = _edge2(g2t, as2v, ad2v, esrc3, edst3, zeros64)

    return _final(acc2_a, acc2_b, g2t, b2.reshape(1, NUM_CLASSES))
